# baseline (speedup 1.0000x reference)
import functools

import jax
import jax.numpy as jnp
from jax import lax
from jax.experimental import pallas as pl
from jax.experimental.pallas import tpu as pltpu
from jax.experimental.pallas import tpu_sc as plsc

_INTERPRET = False
BR = 512
EPS = 1e-5


def _cdiv(a, b):
    return (a + b - 1) // b


def _t0_body(x_ref, wl_ref, wr_ref, ul_ref, uh_ref, r_ref):
    x = x_ref[...]
    u = jnp.dot(x, wl_ref[...], preferred_element_type=jnp.float32, precision=lax.Precision.HIGHEST)
    ul_ref[...] = u[:, :32]
    uh_ref[...] = u[:, 32:]
    r_ref[...] = jnp.dot(x, wr_ref[...], preferred_element_type=jnp.float32, precision=lax.Precision.HIGHEST)


def _ta_body(al_ref, ah_ref, r_ref, deg_ref, b_ref, y_ref, st_ref, acc_ref, *, n_real):
    i = pl.program_id(0)

    @pl.when(i == 0)
    def _():
        acc_ref[...] = jnp.zeros_like(acc_ref)

    a = jnp.concatenate([al_ref[...], ah_ref[...]], axis=1)
    deg = jnp.maximum(deg_ref[...][:, 0:1], 1.0)
    y = a / deg + b_ref[...] + r_ref[...]
    y_ref[...] = y
    row = i * BR + lax.broadcasted_iota(jnp.int32, (BR, 1), 0)
    m = (row < n_real).astype(jnp.float32)
    ym = y * m
    acc_ref[0:1, :] += jnp.sum(ym, axis=0, keepdims=True)
    acc_ref[1:2, :] += jnp.sum(ym * ym, axis=0, keepdims=True)

    @pl.when(i == pl.num_programs(0) - 1)
    def _():
        st_ref[...] = acc_ref[...]


def _bn_from_stats(y, st, g, b, n_real):
    mean = st[0:1, :] / n_real
    var = st[1:2, :] / n_real - mean * mean
    inv = lax.rsqrt(var + EPS)
    return (y - mean) * inv * g + b


def _tb_body(*refs, n_real, has_res, do_mm, h_split):
    it = iter(refs)
    y_ref = next(it)
    st_ref = next(it)
    g_ref = next(it)
    be_ref = next(it)
    hres_ref = next(it) if has_res else None
    wl_ref = next(it) if do_mm else None
    wr_ref = next(it) if do_mm else None
    h = jnp.maximum(_bn_from_stats(y_ref[...], st_ref[...], g_ref[...],
                                   be_ref[...], n_real), 0.0)
    if has_res:
        h = hres_ref[...] + 0.5 * h
    if h_split == "full":
        h_ref = next(it)
        h_ref[...] = h
    elif h_split == "split":
        hl_ref = next(it)
        hh_ref = next(it)
        hl_ref[...] = h[:, :32]
        hh_ref[...] = h[:, 32:]
    if do_mm:
        ul_ref = next(it)
        uh_ref = next(it)
        rr_ref = next(it)
        u = jnp.dot(h, wl_ref[...], preferred_element_type=jnp.float32, precision=lax.Precision.HIGHEST)
        ul_ref[...] = u[:, :32]
        uh_ref[...] = u[:, 32:]
        rr_ref[...] = jnp.dot(h, wr_ref[...], preferred_element_type=jnp.float32, precision=lax.Precision.HIGHEST)


def _bn_masked(v, g, b, mask, count):
    m = jnp.sum(v * mask, axis=0, keepdims=True) / count
    var = jnp.sum(v * v * mask, axis=0, keepdims=True) / count - m * m
    inv = lax.rsqrt(var + EPS)
    return (v - m) * inv * g + b


def _head_body(gsl_ref, gsh_ref, tml_ref, tmh_ref, pgl_ref, pvl_ref, pgh_ref,
               pvh_ref, cnt_ref, adme_ref, gc_ref, bc_ref, w1_ref, b1_ref,
               g1_ref, be1_ref, w2_ref, b2_ref, g2_ref, be2_ref, w3_ref,
               b3_ref, out_ref, scl_ref, sch_ref, *, g_real, n_part):
    scl_ref[...] = tml_ref[...]
    sch_ref[...] = tmh_ref[...]

    def upd(k, _):
        gl = pgl_ref[0, k]
        scl_ref[pl.ds(gl, 1), :] = jnp.maximum(scl_ref[pl.ds(gl, 1), :],
                                               pvl_ref[pl.ds(k, 1), :])
        gh = pgh_ref[0, k]
        sch_ref[pl.ds(gh, 1), :] = jnp.maximum(sch_ref[pl.ds(gh, 1), :],
                                               pvh_ref[pl.ds(k, 1), :])
        return 0

    lax.fori_loop(0, n_part, upd, 0)

    counts = cnt_ref[...][:, 0:1]
    cpos = jnp.maximum(counts, 1.0)
    meanp_lo = gsl_ref[...] / cpos
    meanp_hi = gsh_ref[...] / cpos
    nz = counts > 0
    maxp_lo = jnp.where(nz, scl_ref[...], 0.0)
    maxp_hi = jnp.where(nz, sch_ref[...], 0.0)
    combined = jnp.concatenate(
        [meanp_lo, meanp_hi, maxp_lo, maxp_hi, adme_ref[...]], axis=1)
    gp = combined.shape[0]
    rows = lax.broadcasted_iota(jnp.int32, (gp, 1), 0)
    mask = (rows < g_real).astype(jnp.float32)
    combined = _bn_masked(combined, gc_ref[...], bc_ref[...], mask, g_real)
    combined = combined * mask
    z = jnp.dot(combined, w1_ref[...], preferred_element_type=jnp.float32, precision=lax.Precision.HIGHEST) + b1_ref[...]
    z = jnp.maximum(_bn_masked(z, g1_ref[...], be1_ref[...], mask, g_real), 0.0) * mask
    z = jnp.dot(z, w2_ref[...], preferred_element_type=jnp.float32, precision=lax.Precision.HIGHEST) + b2_ref[...]
    z = jnp.maximum(_bn_masked(z, g2_ref[...], be2_ref[...], mask, g_real), 0.0) * mask
    out_ref[...] = jnp.dot(z, w3_ref[...], preferred_element_type=jnp.float32, precision=lax.Precision.HIGHEST) + b3_ref[...]


def _vspec(c, blk=None):
    b = BR if blk is None else blk
    return pl.BlockSpec((b, c), lambda i: (i, 0))


def _wspec(r, c):
    return pl.BlockSpec((r, c), lambda i: (0, 0))


def _t0(x, wl, wr, n_pad):
    grid = (n_pad // BR,)
    dp = x.shape[1]
    return pl.pallas_call(
        _t0_body,
        grid=grid,
        in_specs=[_vspec(dp), _wspec(dp, 64), _wspec(dp, 64)],
        out_specs=[_vspec(32), _vspec(32), _vspec(64)],
        out_shape=[
            jax.ShapeDtypeStruct((n_pad, 32), jnp.float32),
            jax.ShapeDtypeStruct((n_pad, 32), jnp.float32),
            jax.ShapeDtypeStruct((n_pad, 64), jnp.float32),
        ],
        interpret=_INTERPRET,
    )(x, wl, wr)


def _ta(al, ah, r, deg16, b, n_pad, n_real):
    grid = (n_pad // BR,)
    return pl.pallas_call(
        functools.partial(_ta_body, n_real=n_real),
        grid=grid,
        in_specs=[_vspec(32), _vspec(32), _vspec(64), _vspec(16), _wspec(1, 64)],
        out_specs=[_vspec(64), pl.BlockSpec((8, 64), lambda i: (0, 0))],
        out_shape=[
            jax.ShapeDtypeStruct((n_pad, 64), jnp.float32),
            jax.ShapeDtypeStruct((8, 64), jnp.float32),
        ],
        scratch_shapes=[pltpu.VMEM((8, 64), jnp.float32)],
        interpret=_INTERPRET,
    )(al, ah, r, deg16, b)


def _tb(y, st, g, be, n_pad, n_real, hres=None, wl=None, wr=None,
        h_split="none"):
    grid = (n_pad // BR,)
    has_res = hres is not None
    do_mm = wl is not None
    in_specs = [_vspec(64), pl.BlockSpec((8, 64), lambda i: (0, 0)),
                _wspec(1, 64), _wspec(1, 64)]
    args = [y, st, g, be]
    if has_res:
        in_specs.append(_vspec(64))
        args.append(hres)
    if do_mm:
        in_specs += [_wspec(64, 64), _wspec(64, 64)]
        args += [wl, wr]
    out_specs, out_shape = [], []
    if h_split == "full":
        out_specs.append(_vspec(64))
        out_shape.append(jax.ShapeDtypeStruct((n_pad, 64), jnp.float32))
    elif h_split == "split":
        out_specs += [_vspec(32), _vspec(32)]
        out_shape += [jax.ShapeDtypeStruct((n_pad, 32), jnp.float32)] * 2
    if do_mm:
        out_specs += [_vspec(32), _vspec(32), _vspec(64)]
        out_shape += [jax.ShapeDtypeStruct((n_pad, 32), jnp.float32),
                      jax.ShapeDtypeStruct((n_pad, 32), jnp.float32),
                      jax.ShapeDtypeStruct((n_pad, 64), jnp.float32)]
    return pl.pallas_call(
        functools.partial(_tb_body, n_real=n_real, has_res=has_res,
                          do_mm=do_mm, h_split=h_split),
        grid=grid,
        in_specs=in_specs,
        out_specs=out_specs,
        out_shape=out_shape,
        interpret=_INTERPRET,
    )(*args)


def _head(gsl, gsh, tml, tmh, pgl, pvl, pgh, pvh, cnt16, adme_p, gcp, bcp,
          w1p, b1p, hg1, hbe1, w2, b2, hg2, hbe2, w3p, b3p, g_pad, g_real):
    n_part = pgl.shape[1]
    full = lambda a: pl.BlockSpec(a.shape, lambda: tuple(0 for _ in a.shape))
    smem = lambda a: pl.BlockSpec(a.shape, lambda: tuple(0 for _ in a.shape),
                                  memory_space=pltpu.SMEM)
    args = [gsl, gsh, tml, tmh, pgl, pvl, pgh, pvh, cnt16, adme_p, gcp, bcp,
            w1p, b1p, hg1, hbe1, w2, b2, hg2, hbe2, w3p, b3p]
    in_specs = [full(a) for a in args]
    in_specs[4] = smem(pgl)
    in_specs[6] = smem(pgh)
    return pl.pallas_call(
        functools.partial(_head_body, g_real=g_real, n_part=n_part),
        in_specs=in_specs,
        out_specs=full(jnp.zeros((g_pad, 8))),
        out_shape=jax.ShapeDtypeStruct((g_pad, 8), jnp.float32),
        scratch_shapes=[pltpu.VMEM((g_pad, 32), jnp.float32),
                        pltpu.VMEM((g_pad, 32), jnp.float32)],
        interpret=_INTERPRET,
    )(*args)


def _seg_deg_counts(src, dst, batch, n, n_pad, g, g_pad):
    deg = jax.ops.segment_sum(jnp.ones(dst.shape[0], jnp.float32), dst,
                              num_segments=n)
    cnt = jax.ops.segment_sum(jnp.ones(batch.shape[0], jnp.float32), batch,
                              num_segments=g)
    deg16 = jnp.pad(jnp.tile(deg[:, None], (1, 16)), ((0, n_pad - n), (0, 0)))
    cnt16 = jnp.pad(jnp.tile(cnt[:, None], (1, 16)), ((0, g_pad - g), (0, 0)))
    return deg16, cnt16


def _seg_sum_edges(ul, uh, src, dst, n, n_pad):
    u = jnp.concatenate([ul, uh], axis=1)
    a = jax.ops.segment_sum(u[src], dst, num_segments=n_pad)
    return a[:, :32], a[:, 32:]


def _seg_pool(hl, hh, batch, n, g, g_pad):
    h = jnp.concatenate([hl[:n], hh[:n]], axis=1)
    gs = jax.ops.segment_sum(h, batch, num_segments=g)
    tm = jax.ops.segment_max(h, batch, num_segments=g)
    pad = ((0, g_pad - g), (0, 0))
    gsl = jnp.pad(gs[:, :32], pad)
    gsh = jnp.pad(gs[:, 32:], pad)
    tml = jnp.pad(tm[:, :32], pad, constant_values=-jnp.inf)
    tmh = jnp.pad(tm[:, 32:], pad, constant_values=-jnp.inf)
    npart = 32
    pgl = jnp.full((1, npart), g_pad - 1, jnp.int32)
    pgh = jnp.full((1, npart), g_pad - 1, jnp.int32)
    pvl = jnp.full((npart, 32), -jnp.inf, jnp.float32)
    pvh = jnp.full((npart, 32), -jnp.inf, jnp.float32)
    return gsl, gsh, tml, tmh, pgl, pvl, pgh, pvh


def kernel(x, edge_index, batch, adme_features, W0l, b0, W0r, g0, be0, W1l,
           b1, W1r, g1, be1, W2l, b2, W2r, g2, be2, gc, bc, hW1, hb1, hg1,
           hbe1, hW2, hb2, hg2, hbe2, hW3, hb3):
    n, din = x.shape
    e = edge_index.shape[1]
    g_real, adm = adme_features.shape
    n_pad = _cdiv(n, 4096) * 4096
    g_pad = _cdiv(g_real + 1, 128) * 128
    dp = _cdiv(din, 8) * 8

    x_p = jnp.pad(x, ((0, n_pad - n), (0, dp - din)))
    w0l_p = jnp.pad(W0l, ((0, dp - din), (0, 0)))
    w0r_p = jnp.pad(W0r, ((0, dp - din), (0, 0)))
    src = edge_index[0]
    dst = edge_index[1]

    row = lambda v: v.reshape(1, -1)

    ul, uh, r0 = _t0(x_p, w0l_p, w0r_p, n_pad)
    deg16, cnt16 = _seg_deg_counts(src, dst, batch, n, n_pad, g_real, g_pad)

    al, ah = _seg_sum_edges(ul, uh, src, dst, n, n_pad)
    y1, st1 = _ta(al, ah, r0, deg16, row(b0), n_pad, n)
    h1, u1l, u1h, r1 = _tb(y1, st1, row(g0), row(be0), n_pad, n,
                           wl=W1l, wr=W1r, h_split="full")

    al, ah = _seg_sum_edges(u1l, u1h, src, dst, n, n_pad)
    y2, st2 = _ta(al, ah, r1, deg16, row(b1), n_pad, n)
    u2l, u2h, r2 = _tb(y2, st2, row(g1), row(be1), n_pad, n, hres=h1,
                       wl=W2l, wr=W2r)

    al, ah = _seg_sum_edges(u2l, u2h, src, dst, n, n_pad)
    y3, st3 = _ta(al, ah, r2, deg16, row(b2), n_pad, n)
    h3l, h3h = _tb(y3, st3, row(g2), row(be2), n_pad, n, h_split="split")

    gsl, gsh, tml, tmh, pgl, pvl, pgh, pvh = _seg_pool(h3l, h3h, batch, n,
                                                       g_real, g_pad)

    comb = 2 * 64 + adm
    cpad = _cdiv(comb, 16) * 16
    adme_p = jnp.pad(adme_features, ((0, g_pad - g_real), (0, cpad - 128 - adm)))
    gcp = row(jnp.pad(gc, (0, cpad - comb), constant_values=1.0))
    bcp = row(jnp.pad(bc, (0, cpad - comb)))
    w1p = jnp.pad(hW1, ((0, cpad - comb), (0, 0)))
    w3p = jnp.pad(hW3, ((0, 0), (0, 7)))
    b3p = row(jnp.pad(hb3, (0, 7)))

    out = _head(gsl, gsh, tml, tmh, pgl, pvl, pgh, pvh, cnt16, adme_p, gcp,
                bcp, w1p, row(hb1), row(hg1), row(hbe1), hW2, row(hb2),
                row(hg2), row(hbe2), w3p, b3p, g_pad, g_real)
    return out[:g_real, 0]


# speedup vs baseline: 2.0490x; 2.0490x over previous
import functools

import jax
import jax.numpy as jnp
from jax import lax
from jax.experimental import pallas as pl
from jax.experimental.pallas import tpu as pltpu
from jax.experimental.pallas import tpu_sc as plsc

_INTERPRET = False
BR = 512
EPS = 1e-5


def _cdiv(a, b):
    return (a + b - 1) // b


def _dot(a, b):
    return jnp.dot(a.astype(jnp.bfloat16), b.astype(jnp.bfloat16),
                   preferred_element_type=jnp.float32)


def _t0_body(x_ref, wl_ref, wr_ref, u0_ref, u1_ref, u2_ref, u3_ref, r_ref):
    x = x_ref[...]
    u = _dot(x, wl_ref[...])
    for q, uref in enumerate((u0_ref, u1_ref, u2_ref, u3_ref)):
        uref[...] = u[:, 16 * q:16 * (q + 1)]
    r_ref[...] = _dot(x, wr_ref[...])


def _ta_body(a_ref, r_ref, deg_ref, b_ref, y_ref, st_ref, acc_ref, *, n_real):
    i = pl.program_id(0)

    @pl.when(i == 0)
    def _():
        acc_ref[...] = jnp.zeros_like(acc_ref)

    a = a_ref[...]
    deg = jnp.maximum(deg_ref[...][:, 0:1], 1.0)
    y = a / deg + b_ref[...] + r_ref[...]
    y_ref[...] = y
    row = i * BR + lax.broadcasted_iota(jnp.int32, (BR, 1), 0)
    m = (row < n_real).astype(jnp.float32)
    ym = y * m
    acc_ref[0:1, :] += jnp.sum(ym, axis=0, keepdims=True)
    acc_ref[1:2, :] += jnp.sum(ym * ym, axis=0, keepdims=True)

    @pl.when(i == pl.num_programs(0) - 1)
    def _():
        st_ref[...] = acc_ref[...]


def _bn_from_stats(y, st, g, b, n_real):
    mean = st[0:1, :] / n_real
    var = st[1:2, :] / n_real - mean * mean
    inv = lax.rsqrt(var + EPS)
    return (y - mean) * inv * g + b


def _tb_body(*refs, n_real, has_res, do_mm, h_split):
    it = iter(refs)
    y_ref = next(it)
    st_ref = next(it)
    g_ref = next(it)
    be_ref = next(it)
    hres_ref = next(it) if has_res else None
    wl_ref = next(it) if do_mm else None
    wr_ref = next(it) if do_mm else None
    h = jnp.maximum(_bn_from_stats(y_ref[...], st_ref[...], g_ref[...],
                                   be_ref[...], n_real), 0.0)
    if has_res:
        h = hres_ref[...] + 0.5 * h
    if h_split == "full":
        h_ref = next(it)
        h_ref[...] = h
    elif h_split == "split":
        hl_ref = next(it)
        hh_ref = next(it)
        hl_ref[...] = h[:, :32]
        hh_ref[...] = h[:, 32:]
    if do_mm:
        u0_ref = next(it)
        u1_ref = next(it)
        u2_ref = next(it)
        u3_ref = next(it)
        rr_ref = next(it)
        u = _dot(h, wl_ref[...])
        for q, uref in enumerate((u0_ref, u1_ref, u2_ref, u3_ref)):
            uref[...] = u[:, 16 * q:16 * (q + 1)]
        rr_ref[...] = _dot(h, wr_ref[...])


def _bn_masked(v, g, b, mask, count):
    m = jnp.sum(v * mask, axis=0, keepdims=True) / count
    var = jnp.sum(v * v * mask, axis=0, keepdims=True) / count - m * m
    inv = lax.rsqrt(var + EPS)
    return (v - m) * inv * g + b


def _head_body(gsl_ref, gsh_ref, tml_ref, tmh_ref, pgl_ref, pvl_ref, pgh_ref,
               pvh_ref, cnt_ref, adme_ref, gc_ref, bc_ref, w1_ref, b1_ref,
               g1_ref, be1_ref, w2_ref, b2_ref, g2_ref, be2_ref, w3_ref,
               b3_ref, out_ref, scl_ref, sch_ref, *, g_real, n_part):
    scl_ref[...] = tml_ref[...]
    sch_ref[...] = tmh_ref[...]

    def upd(k, _):
        gl = pgl_ref[0, k]
        scl_ref[pl.ds(gl, 1), :] = jnp.maximum(scl_ref[pl.ds(gl, 1), :],
                                               pvl_ref[pl.ds(k, 1), :])
        gh = pgh_ref[0, k]
        sch_ref[pl.ds(gh, 1), :] = jnp.maximum(sch_ref[pl.ds(gh, 1), :],
                                               pvh_ref[pl.ds(k, 1), :])
        return 0

    lax.fori_loop(0, n_part, upd, 0)

    counts = cnt_ref[...][:, 0:1]
    cpos = jnp.maximum(counts, 1.0)
    meanp_lo = gsl_ref[...] / cpos
    meanp_hi = gsh_ref[...] / cpos
    nz = counts > 0
    maxp_lo = jnp.where(nz, scl_ref[...], 0.0)
    maxp_hi = jnp.where(nz, sch_ref[...], 0.0)
    combined = jnp.concatenate(
        [meanp_lo, meanp_hi, maxp_lo, maxp_hi, adme_ref[...]], axis=1)
    gp = combined.shape[0]
    rows = lax.broadcasted_iota(jnp.int32, (gp, 1), 0)
    mask = (rows < g_real).astype(jnp.float32)
    combined = _bn_masked(combined, gc_ref[...], bc_ref[...], mask, g_real)
    combined = combined * mask
    z = _dot(combined, w1_ref[...]) + b1_ref[...]
    z = jnp.maximum(_bn_masked(z, g1_ref[...], be1_ref[...], mask, g_real), 0.0) * mask
    z = _dot(z, w2_ref[...]) + b2_ref[...]
    z = jnp.maximum(_bn_masked(z, g2_ref[...], be2_ref[...], mask, g_real), 0.0) * mask
    out_ref[...] = _dot(z, w3_ref[...]) + b3_ref[...]


def _vspec(c, blk=None):
    b = BR if blk is None else blk
    return pl.BlockSpec((b, c), lambda i: (i, 0))


def _wspec(r, c):
    return pl.BlockSpec((r, c), lambda i: (0, 0))


def _t0(x, wl, wr, n_pad):
    grid = (n_pad // BR,)
    dp = x.shape[1]
    return pl.pallas_call(
        _t0_body,
        grid=grid,
        in_specs=[_vspec(dp), _wspec(dp, 64), _wspec(dp, 64)],
        out_specs=[_vspec(16)] * 4 + [_vspec(64)],
        out_shape=[jax.ShapeDtypeStruct((n_pad, 16), jnp.float32)] * 4
        + [jax.ShapeDtypeStruct((n_pad, 64), jnp.float32)],
        interpret=_INTERPRET,
    )(x, wl, wr)


def _ta(a, r, deg16, b, n_pad, n_real):
    grid = (n_pad // BR,)
    return pl.pallas_call(
        functools.partial(_ta_body, n_real=n_real),
        grid=grid,
        in_specs=[_vspec(64), _vspec(64), _vspec(16), _wspec(1, 64)],
        out_specs=[_vspec(64), pl.BlockSpec((8, 64), lambda i: (0, 0))],
        out_shape=[
            jax.ShapeDtypeStruct((n_pad, 64), jnp.float32),
            jax.ShapeDtypeStruct((8, 64), jnp.float32),
        ],
        scratch_shapes=[pltpu.VMEM((8, 64), jnp.float32)],
        interpret=_INTERPRET,
    )(a, r, deg16, b)


def _tb(y, st, g, be, n_pad, n_real, hres=None, wl=None, wr=None,
        h_split="none"):
    grid = (n_pad // BR,)
    has_res = hres is not None
    do_mm = wl is not None
    in_specs = [_vspec(64), pl.BlockSpec((8, 64), lambda i: (0, 0)),
                _wspec(1, 64), _wspec(1, 64)]
    args = [y, st, g, be]
    if has_res:
        in_specs.append(_vspec(64))
        args.append(hres)
    if do_mm:
        in_specs += [_wspec(64, 64), _wspec(64, 64)]
        args += [wl, wr]
    out_specs, out_shape = [], []
    if h_split == "full":
        out_specs.append(_vspec(64))
        out_shape.append(jax.ShapeDtypeStruct((n_pad, 64), jnp.float32))
    elif h_split == "split":
        out_specs += [_vspec(32), _vspec(32)]
        out_shape += [jax.ShapeDtypeStruct((n_pad, 32), jnp.float32)] * 2
    if do_mm:
        out_specs += [_vspec(16)] * 4 + [_vspec(64)]
        out_shape += [jax.ShapeDtypeStruct((n_pad, 16), jnp.float32)] * 4
        out_shape += [jax.ShapeDtypeStruct((n_pad, 64), jnp.float32)]
    return pl.pallas_call(
        functools.partial(_tb_body, n_real=n_real, has_res=has_res,
                          do_mm=do_mm, h_split=h_split),
        grid=grid,
        in_specs=in_specs,
        out_specs=out_specs,
        out_shape=out_shape,
        interpret=_INTERPRET,
    )(*args)


def _head(gsl, gsh, tml, tmh, pgl, pvl, pgh, pvh, cnt16, adme_p, gcp, bcp,
          w1p, b1p, hg1, hbe1, w2, b2, hg2, hbe2, w3p, b3p, g_pad, g_real):
    n_part = pgl.shape[1]
    full = lambda a: pl.BlockSpec(a.shape, lambda: tuple(0 for _ in a.shape))
    smem = lambda a: pl.BlockSpec(a.shape, lambda: tuple(0 for _ in a.shape),
                                  memory_space=pltpu.SMEM)
    args = [gsl, gsh, tml, tmh, pgl, pvl, pgh, pvh, cnt16, adme_p, gcp, bcp,
            w1p, b1p, hg1, hbe1, w2, b2, hg2, hbe2, w3p, b3p]
    in_specs = [full(a) for a in args]
    in_specs[4] = smem(pgl)
    in_specs[6] = smem(pgh)
    return pl.pallas_call(
        functools.partial(_head_body, g_real=g_real, n_part=n_part),
        in_specs=in_specs,
        out_specs=full(jnp.zeros((g_pad, 8))),
        out_shape=jax.ShapeDtypeStruct((g_pad, 8), jnp.float32),
        scratch_shapes=[pltpu.VMEM((g_pad, 32), jnp.float32),
                        pltpu.VMEM((g_pad, 32), jnp.float32)],
        interpret=_INTERPRET,
    )(*args)


def _zero_vmem(ref, rows, val=0.0):
    v = jnp.full((16,), val, jnp.float32)

    def zrow(i, _):
        ref[i, 0:16] = v
        ref[i, 16:32] = v
        return 0

    lax.fori_loop(0, rows, zrow, 0)


def _sc_segsum(uq, edges3, n_pad):
    cpw = edges3.shape[1]
    rps = n_pad // 16
    nck = rps // 128
    mesh = plsc.VectorSubcoreMesh(core_axis_name="c", subcore_axis_name="s",
                                  num_cores=2)

    @functools.partial(
        pl.kernel,
        out_type=jax.ShapeDtypeStruct((n_pad, 64), jnp.float32),
        mesh=mesh,
        scratch_types=[
            pltpu.VMEM((2, 128), jnp.int32),
            pltpu.VMEM((128, 16), jnp.float32),
            pltpu.VMEM((128, 16), jnp.float32),
            pltpu.VMEM_SHARED((n_pad, 16), jnp.float32),
            pltpu.SemaphoreType.DMA,
        ],
        compiler_params=pltpu.CompilerParams(use_tc_tiling_on_sc=False),
    )
    def k(u0_hbm, u1_hbm, u2_hbm, u3_hbm, ed_hbm, a_hbm,
          ed_v, rows_v, zbuf_v, acc_sh, sem):
        c = lax.axis_index("c")
        s = lax.axis_index("s")
        r0 = s * rps
        zero = jnp.zeros((16,), jnp.float32)

        def zrow(i, _):
            zbuf_v[i, 0:16] = zero
            return 0

        lax.fori_loop(0, 128, zrow, 0)

        def edge_loop(u_hbm):
            def step(j, _):
                pltpu.sync_copy(ed_hbm.at[s, j], ed_v)
                pltpu.async_copy(u_hbm.at[ed_v.at[0]], rows_v, sem).wait()
                pltpu.sync_copy(rows_v, acc_sh.at[ed_v.at[1]], add=True)
                return 0

            lax.fori_loop(0, cpw, step, 0)

        def wback(q):
            def wb(j, _):
                sl = pl.ds(r0 + j * 128, 128)
                pltpu.sync_copy(acc_sh.at[sl], rows_v)
                pltpu.sync_copy(rows_v, a_hbm.at[sl, pl.ds(16 * q, 16)])
                return 0

            lax.fori_loop(0, nck, wb, 0)

        for p in range(2):

            def zchunk(j, _):
                pltpu.sync_copy(zbuf_v, acc_sh.at[pl.ds(r0 + j * 128, 128)])
                return 0

            lax.fori_loop(0, nck, zchunk, 0)
            plsc.subcore_barrier()

            @pl.when(c == 0)
            def _():
                edge_loop((u0_hbm, u2_hbm)[p])

            @pl.when(c == 1)
            def _():
                edge_loop((u1_hbm, u3_hbm)[p])

            plsc.subcore_barrier()

            @pl.when(c == 0)
            def _():
                wback(2 * p)

            @pl.when(c == 1)
            def _():
                wback(2 * p + 1)

            if p == 0:
                plsc.subcore_barrier()

    return k(*uq, edges3)


def _sc_deg_counts(dst3, batch3, n_pad, g_pad):
    cpw = dst3.shape[1]
    cpb = batch3.shape[1]
    rps_n = n_pad // 16
    rps_g = g_pad // 16
    mesh = plsc.VectorSubcoreMesh(core_axis_name="c", subcore_axis_name="s", num_cores=2)

    @functools.partial(
        pl.kernel,
        out_type=[jax.ShapeDtypeStruct((n_pad, 16), jnp.float32),
                  jax.ShapeDtypeStruct((g_pad, 16), jnp.float32)],
        mesh=mesh,
        scratch_types=[
            pltpu.VMEM((max(cpw, cpb), 128), jnp.int32),
            pltpu.VMEM((128, 16), jnp.float32),
            pltpu.VMEM((128, 16), jnp.float32),
            pltpu.VMEM_SHARED((n_pad, 16), jnp.float32),
        ],
        compiler_params=pltpu.CompilerParams(use_tc_tiling_on_sc=False),
    )
    def k(dst_hbm, b_hbm, deg_hbm, cnt_hbm, idx_v, ones_v, zb_v, acc_sh):
        c = lax.axis_index("c")
        s = lax.axis_index("s")
        one = jnp.ones((16,), jnp.float32)
        zero = jnp.zeros((16,), jnp.float32)

        def fill(i, _):
            ones_v[i, 0:16] = one
            zb_v[i, 0:16] = zero
            return 0

        lax.fori_loop(0, 128, fill, 0)

        def scat(idx_hbm, ncw, rps, out_hbm, ck):
            r0 = s * rps

            def zc(j, _):
                pltpu.sync_copy(zb_v.at[pl.ds(0, ck)],
                                acc_sh.at[pl.ds(r0 + j * ck, ck)])
                return 0

            lax.fori_loop(0, rps // ck, zc, 0)
            plsc.subcore_barrier()
            pltpu.sync_copy(idx_hbm.at[s], idx_v.at[pl.ds(0, ncw)])

            def step(j, _):
                pltpu.sync_copy(ones_v, acc_sh.at[idx_v.at[j]], add=True)
                return 0

            lax.fori_loop(0, ncw, step, 0)
            plsc.subcore_barrier()

            def wb(j, _):
                sl = pl.ds(r0 + j * ck, ck)
                pltpu.sync_copy(acc_sh.at[sl], zb_v.at[pl.ds(0, ck)])
                pltpu.sync_copy(zb_v.at[pl.ds(0, ck)], out_hbm.at[sl])
                return 0

            lax.fori_loop(0, rps // ck, wb, 0)

        @pl.when(c == 0)
        def _():
            scat(dst_hbm, cpw, rps_n, deg_hbm, 128)

        @pl.when(c == 1)
        def _():
            scat(b_hbm, cpb, rps_g, cnt_hbm, 8)

    return k(dst3, batch3)


def _seg_deg_counts(src, dst, batch, n, n_pad, g, g_pad):
    deg = jax.ops.segment_sum(jnp.ones(dst.shape[0], jnp.float32), dst,
                              num_segments=n)
    cnt = jax.ops.segment_sum(jnp.ones(batch.shape[0], jnp.float32), batch,
                              num_segments=g)
    deg16 = jnp.pad(jnp.tile(deg[:, None], (1, 16)), ((0, n_pad - n), (0, 0)))
    cnt16 = jnp.pad(jnp.tile(cnt[:, None], (1, 16)), ((0, g_pad - g), (0, 0)))
    return deg16, cnt16


def _seg_sum_edges(uq, src, dst, n, n_pad):
    u = jnp.concatenate(uq, axis=1)
    return jax.ops.segment_sum(u[src], dst, num_segments=n_pad)


def _seg_pool(hl, hh, batch, n, g, g_pad):
    h = jnp.concatenate([hl[:n], hh[:n]], axis=1)
    gs = jax.ops.segment_sum(h, batch, num_segments=g)
    tm = jax.ops.segment_max(h, batch, num_segments=g)
    pad = ((0, g_pad - g), (0, 0))
    gsl = jnp.pad(gs[:, :32], pad)
    gsh = jnp.pad(gs[:, 32:], pad)
    tml = jnp.pad(tm[:, :32], pad, constant_values=-jnp.inf)
    tmh = jnp.pad(tm[:, 32:], pad, constant_values=-jnp.inf)
    npart = 32
    pgl = jnp.full((1, npart), g_pad - 1, jnp.int32)
    pgh = jnp.full((1, npart), g_pad - 1, jnp.int32)
    pvl = jnp.full((npart, 32), -jnp.inf, jnp.float32)
    pvh = jnp.full((npart, 32), -jnp.inf, jnp.float32)
    return gsl, gsh, tml, tmh, pgl, pvl, pgh, pvh


def kernel(x, edge_index, batch, adme_features, W0l, b0, W0r, g0, be0, W1l,
           b1, W1r, g1, be1, W2l, b2, W2r, g2, be2, gc, bc, hW1, hb1, hg1,
           hbe1, hW2, hb2, hg2, hbe2, hW3, hb3):
    n, din = x.shape
    e = edge_index.shape[1]
    g_real, adm = adme_features.shape
    n_pad = _cdiv(n, 4096) * 4096
    g_pad = _cdiv(g_real + 1, 128) * 128
    dp = _cdiv(din, 8) * 8

    x_p = jnp.pad(x, ((0, n_pad - n), (0, dp - din)))
    w0l_p = jnp.pad(W0l, ((0, dp - din), (0, 0)))
    w0r_p = jnp.pad(W0r, ((0, dp - din), (0, 0)))
    src = edge_index[0]
    dst = edge_index[1]

    row = lambda v: v.reshape(1, -1)

    e_pad = _cdiv(e, 2048) * 2048
    src_p = jnp.concatenate([src, jnp.zeros((e_pad - e,), jnp.int32)])
    dst_p = jnp.concatenate([dst, jnp.full((e_pad - e,), n, jnp.int32)])
    edges3 = jnp.stack([src_p, dst_p]).reshape(2, 16, e_pad // 2048, 128)
    edges3 = edges3.transpose(1, 2, 0, 3)
    batch_p = jnp.concatenate([batch, jnp.full((n_pad - n,), g_real, jnp.int32)])
    batch3 = batch_p.reshape(16, n_pad // 2048, 128)

    u0a, u0b, u0c, u0d, r0 = _t0(x_p, w0l_p, w0r_p, n_pad)
    if _INTERPRET:
        deg16, cnt16 = _seg_deg_counts(src, dst, batch, n, n_pad, g_real, g_pad)
        segsum = lambda uq: _seg_sum_edges(uq, src, dst, n, n_pad)
    else:
        deg16, cnt16 = _seg_deg_counts(src, dst, batch, n, n_pad, g_real, g_pad)
        segsum = lambda uq: _sc_segsum(uq, edges3, n_pad)

    a = segsum((u0a, u0b, u0c, u0d))
    y1, st1 = _ta(a, r0, deg16, row(b0), n_pad, n)
    h1, u1a, u1b, u1c, u1d, r1 = _tb(y1, st1, row(g0), row(be0), n_pad, n,
                                     wl=W1l, wr=W1r, h_split="full")

    a = segsum((u1a, u1b, u1c, u1d))
    y2, st2 = _ta(a, r1, deg16, row(b1), n_pad, n)
    u2a, u2b, u2c, u2d, r2 = _tb(y2, st2, row(g1), row(be1), n_pad, n,
                                 hres=h1, wl=W2l, wr=W2r)

    a = segsum((u2a, u2b, u2c, u2d))
    y3, st3 = _ta(a, r2, deg16, row(b2), n_pad, n)
    h3l, h3h = _tb(y3, st3, row(g2), row(be2), n_pad, n, h_split="split")

    gsl, gsh, tml, tmh, pgl, pvl, pgh, pvh = _seg_pool(h3l, h3h, batch, n,
                                                       g_real, g_pad)

    comb = 2 * 64 + adm
    cpad = _cdiv(comb, 16) * 16
    adme_p = jnp.pad(adme_features, ((0, g_pad - g_real), (0, cpad - 128 - adm)))
    gcp = row(jnp.pad(gc, (0, cpad - comb), constant_values=1.0))
    bcp = row(jnp.pad(bc, (0, cpad - comb)))
    w1p = jnp.pad(hW1, ((0, cpad - comb), (0, 0)))
    w3p = jnp.pad(hW3, ((0, 0), (0, 7)))
    b3p = row(jnp.pad(hb3, (0, 7)))

    out = _head(gsl, gsh, tml, tmh, pgl, pvl, pgh, pvh, cnt16, adme_p, gcp,
                bcp, w1p, row(hb1), row(hg1), row(hbe1), hW2, row(hb2),
                row(hg2), row(hbe2), w3p, b3p, g_pad, g_real)
    return out[:g_real, 0]


# speedup vs baseline: 2.8033x; 1.3681x over previous
import functools

import jax
import jax.numpy as jnp
from jax import lax
from jax.experimental import pallas as pl
from jax.experimental.pallas import tpu as pltpu
from jax.experimental.pallas import tpu_sc as plsc

BR = 512
EPS = 1e-5


def _cdiv(a, b):
    return (a + b - 1) // b


def _dot(a, b):
    return jnp.dot(a.astype(jnp.bfloat16), b.astype(jnp.bfloat16),
                   preferred_element_type=jnp.float32)


def _t0_body(x_ref, wl_ref, wr_ref, u0_ref, u1_ref, u2_ref, u3_ref, r_ref):
    x = x_ref[...]
    u = _dot(x, wl_ref[...])
    for q, uref in enumerate((u0_ref, u1_ref, u2_ref, u3_ref)):
        uref[...] = u[:, 16 * q:16 * (q + 1)]
    r_ref[...] = _dot(x, wr_ref[...])


def _ta_body(a_ref, r_ref, deg_ref, b_ref, y_ref, st_ref, acc_ref, *, n_real):
    i = pl.program_id(0)

    @pl.when(i == 0)
    def _():
        acc_ref[...] = jnp.zeros_like(acc_ref)

    a = a_ref[...]
    deg = jnp.maximum(deg_ref[...][:, 0:1], 1.0)
    y = a / deg + b_ref[...] + r_ref[...]
    y_ref[...] = y
    row = i * BR + lax.broadcasted_iota(jnp.int32, (BR, 1), 0)
    m = (row < n_real).astype(jnp.float32)
    ym = y * m
    acc_ref[0:1, :] += jnp.sum(ym, axis=0, keepdims=True)
    acc_ref[1:2, :] += jnp.sum(ym * ym, axis=0, keepdims=True)

    @pl.when(i == pl.num_programs(0) - 1)
    def _():
        st_ref[...] = acc_ref[...]


def _bn_from_stats(y, st, g, b, n_real):
    mean = st[0:1, :] / n_real
    var = st[1:2, :] / n_real - mean * mean
    inv = lax.rsqrt(var + EPS)
    return (y - mean) * inv * g + b


def _tb_body(*refs, n_real, has_res, do_mm, h_split):
    it = iter(refs)
    y_ref = next(it)
    st_ref = next(it)
    g_ref = next(it)
    be_ref = next(it)
    hres_ref = next(it) if has_res else None
    wl_ref = next(it) if do_mm else None
    wr_ref = next(it) if do_mm else None
    h = jnp.maximum(_bn_from_stats(y_ref[...], st_ref[...], g_ref[...],
                                   be_ref[...], n_real), 0.0)
    if has_res:
        h = hres_ref[...] + 0.5 * h
    if h_split == "full":
        h_ref = next(it)
        h_ref[...] = h
    elif h_split == "split":
        hl_ref = next(it)
        hh_ref = next(it)
        hl_ref[...] = h[:, :32]
        hh_ref[...] = h[:, 32:]
    if do_mm:
        u0_ref = next(it)
        u1_ref = next(it)
        u2_ref = next(it)
        u3_ref = next(it)
        rr_ref = next(it)
        u = _dot(h, wl_ref[...])
        for q, uref in enumerate((u0_ref, u1_ref, u2_ref, u3_ref)):
            uref[...] = u[:, 16 * q:16 * (q + 1)]
        rr_ref[...] = _dot(h, wr_ref[...])


def _bn_masked(v, g, b, mask, count):
    m = jnp.sum(v * mask, axis=0, keepdims=True) / count
    var = jnp.sum(v * v * mask, axis=0, keepdims=True) / count - m * m
    inv = lax.rsqrt(var + EPS)
    return (v - m) * inv * g + b


def _head_body(gsl_ref, gsh_ref, tml_ref, tmh_ref, pgl_ref, pvl_ref, pgh_ref,
               pvh_ref, cnt_ref, adme_ref, gc_ref, bc_ref, w1_ref, b1_ref,
               g1_ref, be1_ref, w2_ref, b2_ref, g2_ref, be2_ref, w3_ref,
               b3_ref, out_ref, scl_ref, sch_ref, *, g_real, n_part):
    scl_ref[...] = tml_ref[...]
    sch_ref[...] = tmh_ref[...]

    def upd(k, _):
        gl = pgl_ref[0, k]
        scl_ref[pl.ds(gl, 1), :] = jnp.maximum(scl_ref[pl.ds(gl, 1), :],
                                               pvl_ref[pl.ds(k, 1), :])
        gh = pgh_ref[0, k]
        sch_ref[pl.ds(gh, 1), :] = jnp.maximum(sch_ref[pl.ds(gh, 1), :],
                                               pvh_ref[pl.ds(k, 1), :])
        return 0

    lax.fori_loop(0, n_part, upd, 0)

    counts = cnt_ref[...][:, 0:1]
    cpos = jnp.maximum(counts, 1.0)
    meanp_lo = gsl_ref[...] / cpos
    meanp_hi = gsh_ref[...] / cpos
    nz = counts > 0
    maxp_lo = jnp.where(nz, scl_ref[...], 0.0)
    maxp_hi = jnp.where(nz, sch_ref[...], 0.0)
    combined = jnp.concatenate(
        [meanp_lo, meanp_hi, maxp_lo, maxp_hi, adme_ref[...]], axis=1)
    gp = combined.shape[0]
    rows = lax.broadcasted_iota(jnp.int32, (gp, 1), 0)
    mask = (rows < g_real).astype(jnp.float32)
    combined = _bn_masked(combined, gc_ref[...], bc_ref[...], mask, g_real)
    combined = combined * mask
    z = _dot(combined, w1_ref[...]) + b1_ref[...]
    z = jnp.maximum(_bn_masked(z, g1_ref[...], be1_ref[...], mask, g_real), 0.0) * mask
    z = _dot(z, w2_ref[...]) + b2_ref[...]
    z = jnp.maximum(_bn_masked(z, g2_ref[...], be2_ref[...], mask, g_real), 0.0) * mask
    out_ref[...] = _dot(z, w3_ref[...]) + b3_ref[...]


def _vspec(c, blk=None):
    b = BR if blk is None else blk
    return pl.BlockSpec((b, c), lambda i: (i, 0))


def _wspec(r, c):
    return pl.BlockSpec((r, c), lambda i: (0, 0))


def _t0(x, wl, wr, n_pad):
    grid = (n_pad // BR,)
    dp = x.shape[1]
    return pl.pallas_call(
        _t0_body,
        grid=grid,
        in_specs=[_vspec(dp), _wspec(dp, 64), _wspec(dp, 64)],
        out_specs=[_vspec(16)] * 4 + [_vspec(64)],
        out_shape=[jax.ShapeDtypeStruct((n_pad, 16), jnp.float32)] * 4
        + [jax.ShapeDtypeStruct((n_pad, 64), jnp.float32)],
    )(x, wl, wr)


def _ta(a, r, deg16, b, n_pad, n_real):
    grid = (n_pad // BR,)
    return pl.pallas_call(
        functools.partial(_ta_body, n_real=n_real),
        grid=grid,
        in_specs=[_vspec(64), _vspec(64), _vspec(16), _wspec(1, 64)],
        out_specs=[_vspec(64), pl.BlockSpec((8, 64), lambda i: (0, 0))],
        out_shape=[
            jax.ShapeDtypeStruct((n_pad, 64), jnp.float32),
            jax.ShapeDtypeStruct((8, 64), jnp.float32),
        ],
        scratch_shapes=[pltpu.VMEM((8, 64), jnp.float32)],
    )(a, r, deg16, b)


def _tb(y, st, g, be, n_pad, n_real, hres=None, wl=None, wr=None,
        h_split="none"):
    grid = (n_pad // BR,)
    has_res = hres is not None
    do_mm = wl is not None
    in_specs = [_vspec(64), pl.BlockSpec((8, 64), lambda i: (0, 0)),
                _wspec(1, 64), _wspec(1, 64)]
    args = [y, st, g, be]
    if has_res:
        in_specs.append(_vspec(64))
        args.append(hres)
    if do_mm:
        in_specs += [_wspec(64, 64), _wspec(64, 64)]
        args += [wl, wr]
    out_specs, out_shape = [], []
    if h_split == "full":
        out_specs.append(_vspec(64))
        out_shape.append(jax.ShapeDtypeStruct((n_pad, 64), jnp.float32))
    elif h_split == "split":
        out_specs += [_vspec(32), _vspec(32)]
        out_shape += [jax.ShapeDtypeStruct((n_pad, 32), jnp.float32)] * 2
    if do_mm:
        out_specs += [_vspec(16)] * 4 + [_vspec(64)]
        out_shape += [jax.ShapeDtypeStruct((n_pad, 16), jnp.float32)] * 4
        out_shape += [jax.ShapeDtypeStruct((n_pad, 64), jnp.float32)]
    return pl.pallas_call(
        functools.partial(_tb_body, n_real=n_real, has_res=has_res,
                          do_mm=do_mm, h_split=h_split),
        grid=grid,
        in_specs=in_specs,
        out_specs=out_specs,
        out_shape=out_shape,
    )(*args)


def _head(gsl, gsh, tml, tmh, pgl, pvl, pgh, pvh, cnt16, adme_p, gcp, bcp,
          w1p, b1p, hg1, hbe1, w2, b2, hg2, hbe2, w3p, b3p, g_pad, g_real):
    n_part = pgl.shape[1]
    full = lambda a: pl.BlockSpec(a.shape, lambda: tuple(0 for _ in a.shape))
    smem = lambda a: pl.BlockSpec(a.shape, lambda: tuple(0 for _ in a.shape),
                                  memory_space=pltpu.SMEM)
    args = [gsl, gsh, tml, tmh, pgl, pvl, pgh, pvh, cnt16, adme_p, gcp, bcp,
            w1p, b1p, hg1, hbe1, w2, b2, hg2, hbe2, w3p, b3p]
    in_specs = [full(a) for a in args]
    in_specs[4] = smem(pgl)
    in_specs[6] = smem(pgh)
    return pl.pallas_call(
        functools.partial(_head_body, g_real=g_real, n_part=n_part),
        in_specs=in_specs,
        out_specs=full(jnp.zeros((g_pad, 8))),
        out_shape=jax.ShapeDtypeStruct((g_pad, 8), jnp.float32),
        scratch_shapes=[pltpu.VMEM((g_pad, 32), jnp.float32),
                        pltpu.VMEM((g_pad, 32), jnp.float32)],
    )(*args)


def _zero_vmem(ref, rows, val=0.0):
    v = jnp.full((16,), val, jnp.float32)

    def zrow(i, _):
        ref[i, 0:16] = v
        ref[i, 16:32] = v
        return 0

    lax.fori_loop(0, rows, zrow, 0)


def _sc_segsum(uq, edges3, n_pad):
    cpw = edges3.shape[1]
    rps = n_pad // 16
    nck = rps // 128
    mesh = plsc.VectorSubcoreMesh(core_axis_name="c", subcore_axis_name="s",
                                  num_cores=2)

    @functools.partial(
        pl.kernel,
        out_type=jax.ShapeDtypeStruct((n_pad, 64), jnp.float32),
        mesh=mesh,
        scratch_types=[
            pltpu.VMEM((2, 128), jnp.int32),
            pltpu.VMEM((2, 128), jnp.int32),
            pltpu.VMEM((128, 16), jnp.float32),
            pltpu.VMEM((128, 16), jnp.float32),
            pltpu.VMEM((128, 16), jnp.float32),
            pltpu.VMEM_SHARED((n_pad, 16), jnp.float32),
            pltpu.SemaphoreType.DMA,
            pltpu.SemaphoreType.DMA,
        ],
        compiler_params=pltpu.CompilerParams(use_tc_tiling_on_sc=False),
    )
    def k(u0_hbm, u1_hbm, u2_hbm, u3_hbm, ed_hbm, a_hbm,
          ed_v, ed_w, rows_v, rows_w, zbuf_v, acc_sh, sem, sem2):
        c = lax.axis_index("c")
        s = lax.axis_index("s")
        r0 = s * rps
        zero = jnp.zeros((16,), jnp.float32)

        def zrow(i, _):
            zbuf_v[i, 0:16] = zero
            return 0

        lax.fori_loop(0, 128, zrow, 0)

        def edge_loop(u_hbm):
            pltpu.sync_copy(ed_hbm.at[s, 0], ed_v)
            pltpu.async_copy(u_hbm.at[ed_v.at[0]], rows_v, sem)

            def step(j2, _):
                j = 2 * j2
                pltpu.sync_copy(ed_hbm.at[s, j + 1], ed_w)
                pltpu.async_copy(u_hbm.at[ed_w.at[0]], rows_w, sem2)
                pltpu.make_async_copy(u_hbm.at[ed_v.at[0]], rows_v, sem).wait()
                pltpu.sync_copy(rows_v, acc_sh.at[ed_v.at[1]], add=True)

                @pl.when(j + 2 < cpw)
                def _():
                    pltpu.sync_copy(ed_hbm.at[s, j + 2], ed_v)
                    pltpu.async_copy(u_hbm.at[ed_v.at[0]], rows_v, sem)

                pltpu.make_async_copy(u_hbm.at[ed_w.at[0]], rows_w, sem2).wait()
                pltpu.sync_copy(rows_w, acc_sh.at[ed_w.at[1]], add=True)
                return 0

            lax.fori_loop(0, cpw // 2, step, 0)

        def wback(q):
            def wb(j, _):
                sl = pl.ds(r0 + j * 128, 128)
                pltpu.sync_copy(acc_sh.at[sl], rows_v)
                pltpu.sync_copy(rows_v, a_hbm.at[sl, pl.ds(16 * q, 16)])
                return 0

            lax.fori_loop(0, nck, wb, 0)

        for p in range(2):

            def zchunk(j, _):
                pltpu.sync_copy(zbuf_v, acc_sh.at[pl.ds(r0 + j * 128, 128)])
                return 0

            lax.fori_loop(0, nck, zchunk, 0)
            plsc.subcore_barrier()

            @pl.when(c == 0)
            def _():
                edge_loop((u0_hbm, u2_hbm)[p])

            @pl.when(c == 1)
            def _():
                edge_loop((u1_hbm, u3_hbm)[p])

            plsc.subcore_barrier()

            @pl.when(c == 0)
            def _():
                wback(2 * p)

            @pl.when(c == 1)
            def _():
                wback(2 * p + 1)

            if p == 0:
                plsc.subcore_barrier()

    return k(*uq, edges3)


def _sc_deg_counts(dst3, batch3, n_pad, g_pad):
    cpw = dst3.shape[1]
    cpb = batch3.shape[1]
    rps_n = n_pad // 16
    rps_g = g_pad // 16
    mesh = plsc.VectorSubcoreMesh(core_axis_name="c", subcore_axis_name="s", num_cores=2)

    @functools.partial(
        pl.kernel,
        out_type=[jax.ShapeDtypeStruct((n_pad, 16), jnp.float32),
                  jax.ShapeDtypeStruct((g_pad, 16), jnp.float32)],
        mesh=mesh,
        scratch_types=[
            pltpu.VMEM((max(cpw, cpb), 128), jnp.int32),
            pltpu.VMEM((128, 16), jnp.float32),
            pltpu.VMEM((128, 16), jnp.float32),
            pltpu.VMEM_SHARED((n_pad, 16), jnp.float32),
        ],
        compiler_params=pltpu.CompilerParams(use_tc_tiling_on_sc=False),
    )
    def k(dst_hbm, b_hbm, deg_hbm, cnt_hbm, idx_v, ones_v, zb_v, acc_sh):
        c = lax.axis_index("c")
        s = lax.axis_index("s")
        one = jnp.ones((16,), jnp.float32)
        zero = jnp.zeros((16,), jnp.float32)

        def fill(i, _):
            ones_v[i, 0:16] = one
            zb_v[i, 0:16] = zero
            return 0

        lax.fori_loop(0, 128, fill, 0)

        def scat(idx_hbm, ncw, rps, out_hbm, ck):
            r0 = s * rps

            def zc(j, _):
                pltpu.sync_copy(zb_v.at[pl.ds(0, ck)],
                                acc_sh.at[pl.ds(r0 + j * ck, ck)])
                return 0

            lax.fori_loop(0, rps // ck, zc, 0)
            plsc.subcore_barrier()
            pltpu.sync_copy(idx_hbm.at[s], idx_v.at[pl.ds(0, ncw)])

            def step(j, _):
                pltpu.sync_copy(ones_v, acc_sh.at[idx_v.at[j]], add=True)
                return 0

            lax.fori_loop(0, ncw, step, 0)
            plsc.subcore_barrier()

            def wb(j, _):
                sl = pl.ds(r0 + j * ck, ck)
                pltpu.sync_copy(acc_sh.at[sl], zb_v.at[pl.ds(0, ck)])
                pltpu.sync_copy(zb_v.at[pl.ds(0, ck)], out_hbm.at[sl])
                return 0

            lax.fori_loop(0, rps // ck, wb, 0)

        @pl.when(c == 0)
        def _():
            scat(dst_hbm, cpw, rps_n, deg_hbm, 128)

        @pl.when(c == 1)
        def _():
            scat(b_hbm, cpb, rps_g, cnt_hbm, 8)

    return k(dst3, batch3)


def _seg_deg_counts(src, dst, batch, n, n_pad, g, g_pad):
    deg = jax.ops.segment_sum(jnp.ones(dst.shape[0], jnp.float32), dst,
                              num_segments=n)
    cnt = jax.ops.segment_sum(jnp.ones(batch.shape[0], jnp.float32), batch,
                              num_segments=g)
    deg16 = jnp.pad(jnp.tile(deg[:, None], (1, 16)), ((0, n_pad - n), (0, 0)))
    cnt16 = jnp.pad(jnp.tile(cnt[:, None], (1, 16)), ((0, g_pad - g), (0, 0)))
    return deg16, cnt16


def _seg_sum_edges(uq, src, dst, n, n_pad):
    u = jnp.concatenate(uq, axis=1)
    return jax.ops.segment_sum(u[src], dst, num_segments=n_pad)


def _seg_pool(hl, hh, batch, n, g, g_pad):
    h = jnp.concatenate([hl[:n], hh[:n]], axis=1)
    gs = jax.ops.segment_sum(h, batch, num_segments=g)
    tm = jax.ops.segment_max(h, batch, num_segments=g)
    pad = ((0, g_pad - g), (0, 0))
    gsl = jnp.pad(gs[:, :32], pad)
    gsh = jnp.pad(gs[:, 32:], pad)
    tml = jnp.pad(tm[:, :32], pad, constant_values=-jnp.inf)
    tmh = jnp.pad(tm[:, 32:], pad, constant_values=-jnp.inf)
    npart = 32
    pgl = jnp.full((1, npart), g_pad - 1, jnp.int32)
    pgh = jnp.full((1, npart), g_pad - 1, jnp.int32)
    pvl = jnp.full((npart, 32), -jnp.inf, jnp.float32)
    pvh = jnp.full((npart, 32), -jnp.inf, jnp.float32)
    return gsl, gsh, tml, tmh, pgl, pvl, pgh, pvh


def kernel(x, edge_index, batch, adme_features, W0l, b0, W0r, g0, be0, W1l,
           b1, W1r, g1, be1, W2l, b2, W2r, g2, be2, gc, bc, hW1, hb1, hg1,
           hbe1, hW2, hb2, hg2, hbe2, hW3, hb3):
    n, din = x.shape
    e = edge_index.shape[1]
    g_real, adm = adme_features.shape
    n_pad = _cdiv(n, 4096) * 4096
    g_pad = _cdiv(g_real + 1, 128) * 128
    dp = _cdiv(din, 8) * 8

    x_p = jnp.pad(x, ((0, n_pad - n), (0, dp - din)))
    w0l_p = jnp.pad(W0l, ((0, dp - din), (0, 0)))
    w0r_p = jnp.pad(W0r, ((0, dp - din), (0, 0)))
    src = edge_index[0]
    dst = edge_index[1]

    row = lambda v: v.reshape(1, -1)

    e_pad = _cdiv(e, 4096) * 4096
    src_p = jnp.concatenate([src, jnp.zeros((e_pad - e,), jnp.int32)])
    dst_p = jnp.concatenate([dst, jnp.full((e_pad - e,), n, jnp.int32)])
    edges3 = jnp.stack([src_p, dst_p]).reshape(2, 16, e_pad // 2048, 128)
    edges3 = edges3.transpose(1, 2, 0, 3)
    batch_p = jnp.concatenate([batch, jnp.full((n_pad - n,), g_real, jnp.int32)])
    batch3 = batch_p.reshape(16, n_pad // 2048, 128)

    u0a, u0b, u0c, u0d, r0 = _t0(x_p, w0l_p, w0r_p, n_pad)
    deg16, cnt16 = _seg_deg_counts(src, dst, batch, n, n_pad, g_real, g_pad)
    segsum = lambda uq: _sc_segsum(uq, edges3, n_pad)

    a = segsum((u0a, u0b, u0c, u0d))
    y1, st1 = _ta(a, r0, deg16, row(b0), n_pad, n)
    h1, u1a, u1b, u1c, u1d, r1 = _tb(y1, st1, row(g0), row(be0), n_pad, n,
                                     wl=W1l, wr=W1r, h_split="full")

    a = segsum((u1a, u1b, u1c, u1d))
    y2, st2 = _ta(a, r1, deg16, row(b1), n_pad, n)
    u2a, u2b, u2c, u2d, r2 = _tb(y2, st2, row(g1), row(be1), n_pad, n,
                                 hres=h1, wl=W2l, wr=W2r)

    a = segsum((u2a, u2b, u2c, u2d))
    y3, st3 = _ta(a, r2, deg16, row(b2), n_pad, n)
    h3l, h3h = _tb(y3, st3, row(g2), row(be2), n_pad, n, h_split="split")

    gsl, gsh, tml, tmh, pgl, pvl, pgh, pvh = _seg_pool(h3l, h3h, batch, n,
                                                       g_real, g_pad)

    comb = 2 * 64 + adm
    cpad = _cdiv(comb, 16) * 16
    adme_p = jnp.pad(adme_features, ((0, g_pad - g_real), (0, cpad - 128 - adm)))
    gcp = row(jnp.pad(gc, (0, cpad - comb), constant_values=1.0))
    bcp = row(jnp.pad(bc, (0, cpad - comb)))
    w1p = jnp.pad(hW1, ((0, cpad - comb), (0, 0)))
    w3p = jnp.pad(hW3, ((0, 0), (0, 7)))
    b3p = row(jnp.pad(hb3, (0, 7)))

    out = _head(gsl, gsh, tml, tmh, pgl, pvl, pgh, pvh, cnt16, adme_p, gcp,
                bcp, w1p, row(hb1), row(hg1), row(hbe1), hW2, row(hb2),
                row(hg2), row(hbe2), w3p, b3p, g_pad, g_real)
    return out[:g_real, 0]
